# baseline (speedup 1.0000x reference)
import functools
import jax
import jax.numpy as jnp
from jax import lax
from jax.experimental import pallas as pl
from jax.experimental.pallas import tpu as pltpu
from jax.experimental.pallas import tpu_sc as plsc

_NUM_FIELDS = 26
_VOCAB = 100000
_DIM = 16
_BATCH = 16384
_DENSE = 13

_NC, _NS = 2, 16
_NW = _NC * _NS
_B_PER_W = _BATCH // _NW
_CHUNK = 128
_NCHUNK = _B_PER_W // _CHUNK
_IDX_N = _NUM_FIELDS * _B_PER_W


def _body(sparse_hbm, tab_hbm, out_hbm, idx_v, g_v, sem):
    wid = lax.axis_index("s") * _NC + lax.axis_index("c")
    base = wid * _B_PER_W

    stage = [
        pltpu.async_copy(
            sparse_hbm.at[pl.ds(i * _BATCH + base, _B_PER_W)],
            idx_v.at[pl.ds(i * _B_PER_W, _B_PER_W)],
            sem,
        )
        for i in range(_NUM_FIELDS)
    ]
    for cp in stage:
        cp.wait()

    def add_off(i, _):
        off = jnp.full((16,), i * _VOCAB, dtype=jnp.int32)

        def add_vec(v, _):
            s = pl.ds(i * _B_PER_W + v * 16, 16)
            idx_v[s] = idx_v[s] + off
            return _

        lax.fori_loop(0, _B_PER_W // 16, add_vec, None)
        return _

    lax.fori_loop(0, _NUM_FIELDS, add_off, None)

    def do_chunk(c, _):
        row0 = base + c * _CHUNK
        gathers = [
            pltpu.async_copy(
                tab_hbm.at[idx_v.at[pl.ds(i * _B_PER_W + c * _CHUNK, _CHUNK)]],
                g_v.at[i],
                sem,
            )
            for i in range(_NUM_FIELDS)
        ]
        for cp in gathers:
            cp.wait()
        writes = [
            pltpu.async_copy(
                g_v.at[i],
                out_hbm.at[i, pl.ds(row0, _CHUNK), :],
                sem,
            )
            for i in range(_NUM_FIELDS)
        ]
        for cp in writes:
            cp.wait()
        return _

    lax.fori_loop(0, _NCHUNK, do_chunk, None)


@jax.jit
def kernel(dense, sparse, tables):
    tab_flat = tables.reshape(_NUM_FIELDS * _VOCAB, _DIM)
    sparse_tf = sparse.T.reshape(-1)
    mesh = plsc.VectorSubcoreMesh(core_axis_name="c", subcore_axis_name="s")
    run = pl.kernel(
        _body,
        out_type=jax.ShapeDtypeStruct((_NUM_FIELDS, _BATCH, _DIM), jnp.float32),
        mesh=mesh,
        scratch_types=[
            pltpu.VMEM((_IDX_N,), jnp.int32),
            pltpu.VMEM((_NUM_FIELDS, _CHUNK, _DIM), jnp.float32),
            pltpu.SemaphoreType.DMA,
        ],
        compiler_params=pltpu.CompilerParams(use_tc_tiling_on_sc=False),
    )
    embeds = run(sparse_tf, tab_flat)
    sparse_feats = embeds.transpose(1, 0, 2).reshape(_BATCH, _NUM_FIELDS * _DIM)
    return jnp.concatenate([dense, sparse_feats], axis=-1)


# speedup vs baseline: 1.0882x; 1.0882x over previous
import functools
import jax
import jax.numpy as jnp
from jax import lax
from jax.experimental import pallas as pl
from jax.experimental.pallas import tpu as pltpu
from jax.experimental.pallas import tpu_sc as plsc

_NUM_FIELDS = 26
_VOCAB = 100000
_DIM = 16
_BATCH = 16384
_DENSE = 13

_NC, _NS = 2, 16
_NW = _NC * _NS
_B_PER_W = _BATCH // _NW
_IDX_N = _NUM_FIELDS * _B_PER_W
_PERIOD = 208
_GBLK = 104
_CHUNK_ROWS = 128
_CHUNK_IDX = _CHUNK_ROWS * _NUM_FIELDS
_G_PER_CHUNK = _CHUNK_IDX // _GBLK
_NCHUNK = _B_PER_W // _CHUNK_ROWS


def _body(sparse_hbm, tab_hbm, out_hbm, idx_v, offs_v, asm_v, sem):
    wid = lax.axis_index("s") * _NC + lax.axis_index("c")
    base = wid * _B_PER_W

    stage = pltpu.async_copy(
        sparse_hbm.at[pl.ds(wid * _IDX_N, _IDX_N)], idx_v, sem
    )

    for v in range(_PERIOD // 16):
        lanes = lax.iota(jnp.int32, 16) + (v * 16)
        offs_v[pl.ds(v * 16, 16)] = lax.rem(lanes, _NUM_FIELDS) * _VOCAB

    stage.wait()

    def add_off(g, _):
        for v in range(_PERIOD // 16):
            s = pl.ds(g * _PERIOD + v * 16, 16)
            idx_v[s] = idx_v[s] + offs_v[pl.ds(v * 16, 16)]
        return _

    lax.fori_loop(0, _IDX_N // _PERIOD, add_off, None)

    def do_chunk(c, _):
        copies = [
            pltpu.async_copy(
                tab_hbm.at[idx_v.at[pl.ds(c * _CHUNK_IDX + k * _GBLK, _GBLK)]],
                asm_v.at[pl.ds(k * _GBLK, _GBLK), :],
                sem,
            )
            for k in range(_G_PER_CHUNK)
        ]
        for cp in copies:
            cp.wait()
        row0 = (base + c * _CHUNK_ROWS) * _NUM_FIELDS
        pltpu.sync_copy(asm_v, out_hbm.at[pl.ds(row0, _CHUNK_IDX), :])
        return _

    lax.fori_loop(0, _NCHUNK, do_chunk, None)


@jax.jit
def kernel(dense, sparse, tables):
    tab_flat = tables.reshape(_NUM_FIELDS * _VOCAB, _DIM)
    sparse_flat = sparse.reshape(-1)
    mesh = plsc.VectorSubcoreMesh(core_axis_name="c", subcore_axis_name="s")
    run = pl.kernel(
        _body,
        out_type=jax.ShapeDtypeStruct((_BATCH * _NUM_FIELDS, _DIM), jnp.float32),
        mesh=mesh,
        scratch_types=[
            pltpu.VMEM((_IDX_N,), jnp.int32),
            pltpu.VMEM((_PERIOD,), jnp.int32),
            pltpu.VMEM((_CHUNK_IDX, _DIM), jnp.float32),
            pltpu.SemaphoreType.DMA,
        ],
        compiler_params=pltpu.CompilerParams(use_tc_tiling_on_sc=False),
    )
    embeds = run(sparse_flat, tab_flat)
    sparse_feats = embeds.reshape(_BATCH, _NUM_FIELDS * _DIM)
    return jnp.concatenate([dense, sparse_feats], axis=-1)


# speedup vs baseline: 1.0912x; 1.0027x over previous
import functools
import jax
import jax.numpy as jnp
from jax import lax
from jax.experimental import pallas as pl
from jax.experimental.pallas import tpu as pltpu
from jax.experimental.pallas import tpu_sc as plsc

_NUM_FIELDS = 26
_VOCAB = 100000
_DIM = 16
_BATCH = 16384
_DENSE = 13

_NC, _NS = 2, 16
_NW = _NC * _NS
_B_PER_W = _BATCH // _NW
_IDX_N = _NUM_FIELDS * _B_PER_W
_PERIOD = 208
_CHUNK_ROWS = 64
_CHUNK_IDX = _CHUNK_ROWS * _NUM_FIELDS
_NCHUNK = _B_PER_W // _CHUNK_ROWS


def _body(sparse_hbm, tab_hbm, out_hbm, idx_v, offs_v, asm0, asm1, sem0, sem1,
          wsem0, wsem1):
    wid = lax.axis_index("s") * _NC + lax.axis_index("c")
    base = wid * _B_PER_W

    stage = pltpu.async_copy(
        sparse_hbm.at[pl.ds(wid * _IDX_N, _IDX_N)], idx_v, sem0
    )

    for v in range(_PERIOD // 16):
        lanes = lax.iota(jnp.int32, 16) + (v * 16)
        offs_v[pl.ds(v * 16, 16)] = lax.rem(lanes, _NUM_FIELDS) * _VOCAB

    stage.wait()

    def add_off(g, _):
        for v in range(_PERIOD // 16):
            s = pl.ds(g * _PERIOD + v * 16, 16)
            idx_v[s] = idx_v[s] + offs_v[pl.ds(v * 16, 16)]
        return _

    lax.fori_loop(0, _IDX_N // _PERIOD, add_off, None)

    asm = (asm0, asm1)
    gsem = (sem0, sem1)
    wsem = (wsem0, wsem1)

    def gather(c):
        return pltpu.async_copy(
            tab_hbm.at[idx_v.at[pl.ds(c * _CHUNK_IDX, _CHUNK_IDX)]],
            asm[c % 2],
            gsem[c % 2],
        )

    def write(c):
        row0 = (base + c * _CHUNK_ROWS) * _NUM_FIELDS
        return pltpu.async_copy(
            asm[c % 2], out_hbm.at[pl.ds(row0, _CHUNK_IDX), :], wsem[c % 2]
        )

    gathers = [None] * _NCHUNK
    writes = [None] * _NCHUNK
    gathers[0] = gather(0)
    for c in range(_NCHUNK):
        if c >= 1:
            writes[c - 1].wait()
        if c + 1 < _NCHUNK:
            gathers[c + 1] = gather(c + 1)
        gathers[c].wait()
        writes[c] = write(c)
    writes[_NCHUNK - 1].wait()


@jax.jit
def kernel(dense, sparse, tables):
    tab_flat = tables.reshape(_NUM_FIELDS * _VOCAB, _DIM)
    sparse_flat = sparse.reshape(-1)
    mesh = plsc.VectorSubcoreMesh(core_axis_name="c", subcore_axis_name="s")
    run = pl.kernel(
        _body,
        out_type=jax.ShapeDtypeStruct((_BATCH * _NUM_FIELDS, _DIM), jnp.float32),
        mesh=mesh,
        scratch_types=[
            pltpu.VMEM((_IDX_N,), jnp.int32),
            pltpu.VMEM((_PERIOD,), jnp.int32),
            pltpu.VMEM((_CHUNK_IDX, _DIM), jnp.float32),
            pltpu.VMEM((_CHUNK_IDX, _DIM), jnp.float32),
            pltpu.SemaphoreType.DMA,
            pltpu.SemaphoreType.DMA,
            pltpu.SemaphoreType.DMA,
            pltpu.SemaphoreType.DMA,
        ],
        compiler_params=pltpu.CompilerParams(use_tc_tiling_on_sc=False),
    )
    embeds = run(sparse_flat, tab_flat)
    sparse_feats = embeds.reshape(_BATCH, _NUM_FIELDS * _DIM)
    return jnp.concatenate([dense, sparse_feats], axis=-1)
